# Initial kernel scaffold; baseline (speedup 1.0000x reference)
import jax
import jax.numpy as jnp
from jax.experimental import pallas as pl


def kernel(x, enc_w1, enc_b1, enc_w2, enc_b2, kp_w1, kp_b1, kp_w2, kp_b2, kp_w3, kp_b3, k_scale):
    raise NotImplementedError("write your pallas kernel here")


# speedup vs baseline: 2.0107x; 2.0107x over previous
import functools

import jax
import jax.numpy as jnp
import numpy as np
from jax import lax
from jax.experimental import pallas as pl
from jax.experimental.pallas import tpu as pltpu
from jax.experimental.pallas import tpu_sc as plsc

INPUT_DIM_ = 1024
N_HDIM_ = 512
QDIM_ = 4096
BATCH_ = 4096

_BM = 256
_GRID = BATCH_ // _BM
_PREC = lax.Precision.DEFAULT

_NC = 2
_NS = 16
_NW = _NC * _NS
_RPW = BATCH_ // _NW
_L = 16
_NCH = QDIM_ // _L
_SIGN = np.int32(-(2**31))


def _dot(a, b):
    return lax.dot_general(a, b, (((1,), (0,)), ((), ())),
                           precision=_PREC,
                           preferred_element_type=jnp.float32)


def _tc_body(x_ref, w1_ref, b1_ref, w2_ref, b2_ref, kpw1_ref, kpb1_ref,
             kpw2_ref, kpb2_ref, kpw3_ref, kpb3_ref, ksc_ref,
             logits_ref, k_ref, n_ref):
    x = x_ref[...]
    h = jnp.maximum(_dot(x, w1_ref[...]) + b1_ref[...][None, :], 0.0)
    logits = _dot(h, w2_ref[...]) + b2_ref[...][None, :]
    logits_ref[...] = logits
    h1 = jnp.maximum(_dot(x, kpw1_ref[0:INPUT_DIM_, :])
                     + _dot(logits, kpw1_ref[INPUT_DIM_:, :])
                     + kpb1_ref[...][None, :], 0.0)
    h2 = jnp.maximum(_dot(h1, kpw2_ref[...]) + kpb2_ref[...][None, :], 0.0)
    z = jnp.sum(h2 * kpw3_ref[...], axis=-1, keepdims=True) + kpb3_ref[...]
    k = jax.nn.sigmoid(z) * float(QDIM_)
    k = jnp.clip(k * jax.nn.sigmoid(ksc_ref[...]) * 2.0, 1.0, float(QDIM_))
    k_ref[...] = k[:, 0]
    n_ref[...] = jnp.ceil(k[:, 0]).astype(jnp.int32)


def _tc_call(x, enc_w1, enc_b1, enc_w2, enc_b2, kp_w1, kp_b1, kp_w2, kp_b2,
             kp_w3_row, kp_b3s, k_scales):
    return pl.pallas_call(
        _tc_body,
        grid=(_GRID,),
        in_specs=[
            pl.BlockSpec((_BM, INPUT_DIM_), lambda i: (i, 0)),
            pl.BlockSpec((INPUT_DIM_, N_HDIM_), lambda i: (0, 0)),
            pl.BlockSpec((N_HDIM_,), lambda i: (0,)),
            pl.BlockSpec((N_HDIM_, QDIM_), lambda i: (0, 0)),
            pl.BlockSpec((QDIM_,), lambda i: (0,)),
            pl.BlockSpec((INPUT_DIM_ + QDIM_, N_HDIM_), lambda i: (0, 0)),
            pl.BlockSpec((N_HDIM_,), lambda i: (0,)),
            pl.BlockSpec((N_HDIM_, N_HDIM_), lambda i: (0, 0)),
            pl.BlockSpec((N_HDIM_,), lambda i: (0,)),
            pl.BlockSpec((1, N_HDIM_), lambda i: (0, 0)),
            pl.BlockSpec((1, 1), lambda i: (0, 0)),
            pl.BlockSpec((1, 1), lambda i: (0, 0)),
        ],
        out_specs=[
            pl.BlockSpec((_BM, QDIM_), lambda i: (i, 0)),
            pl.BlockSpec((_BM,), lambda i: (i,)),
            pl.BlockSpec((_BM,), lambda i: (i,)),
        ],
        out_shape=[
            jax.ShapeDtypeStruct((BATCH_, QDIM_), jnp.float32),
            jax.ShapeDtypeStruct((BATCH_,), jnp.float32),
            jax.ShapeDtypeStruct((BATCH_,), jnp.int32),
        ],
    )(x, enc_w1, enc_b1, enc_w2, enc_b2, kp_w1, kp_b1, kp_w2, kp_b2,
      kp_w3_row, kp_b3s, k_scales)


def _sc_body(logits_hbm, n_hbm, out_hbm, row_v, s_v, out_v, n_v):
    cid = lax.axis_index("c")
    sid = lax.axis_index("s")
    wid = sid * _NC + cid
    base = wid * _RPW
    pltpu.sync_copy(n_hbm.at[pl.ds(base, _RPW)], n_v)

    def count_ge(thr):
        def cc(ci, acc):
            sv = s_v[pl.ds(ci * _L, _L)]
            return acc + jnp.where(sv >= thr, 1, 0).astype(jnp.int32)
        acc = lax.fori_loop(0, _NCH, cc, jnp.zeros((_L,), jnp.int32))
        return jnp.sum(acc)

    def count_gt(thr):
        def cc(ci, acc):
            sv = s_v[pl.ds(ci * _L, _L)]
            return acc + jnp.where(sv > thr, 1, 0).astype(jnp.int32)
        acc = lax.fori_loop(0, _NCH, cc, jnp.zeros((_L,), jnp.int32))
        return jnp.sum(acc)

    def do_row(r, carry):
        row = base + r
        pltpu.sync_copy(logits_hbm.at[row], row_v)
        nchunk = n_v[pl.ds((r // _L) * _L, _L)]
        lane = lax.broadcasted_iota(jnp.int32, (_L,), 0)
        n = jnp.sum(jnp.where(lane == r % _L, nchunk, 0))

        def map_chunk(ci, c):
            f = row_v[pl.ds(ci * _L, _L)]
            b = plsc.bitcast(f, jnp.int32)
            sv = jnp.where(b < 0, -(b & jnp.int32(0x7FFFFFFF)), b)
            s_v[pl.ds(ci * _L, _L)] = sv
            return c
        lax.fori_loop(0, _NCH, map_chunk, 0)

        def bit_step(i, t_u):
            bit = 31 - i
            t2 = t_u | (jnp.int32(1) << bit)
            cnt = count_ge(t2 ^ _SIGN)
            return jnp.where(cnt >= n, t2, t_u)
        t_u = lax.fori_loop(0, 32, bit_step, jnp.int32(0))
        thr = t_u ^ _SIGN

        c_gt = count_gt(thr)
        rneed = n - c_gt

        def emit_chunk(ci, seen):
            sv = s_v[pl.ds(ci * _L, _L)]
            m_gt = sv > thr
            m_eq = sv == thr
            eq = jnp.where(m_eq, 1, 0).astype(jnp.int32)
            pref = plsc.cumsum(eq)
            take = m_gt | (m_eq & ((seen + pref) <= rneed))
            out_v[pl.ds(ci * _L, _L)] = jnp.where(take, 1.0, 0.0).astype(jnp.float32)
            return seen + jnp.sum(eq)
        lax.fori_loop(0, _NCH, emit_chunk, jnp.int32(0))
        pltpu.sync_copy(out_v, out_hbm.at[row])
        return carry
    lax.fori_loop(0, _RPW, do_row, 0)


@functools.cache
def _sc_mask_call():
    return pl.kernel(
        _sc_body,
        out_type=jax.ShapeDtypeStruct((BATCH_, QDIM_), jnp.float32),
        mesh=plsc.VectorSubcoreMesh(core_axis_name="c", subcore_axis_name="s",
                                    num_cores=_NC, num_subcores=_NS),
        compiler_params=pltpu.CompilerParams(needs_layout_passes=False),
        scratch_types=[
            pltpu.VMEM((QDIM_,), jnp.float32),
            pltpu.VMEM((QDIM_,), jnp.int32),
            pltpu.VMEM((QDIM_,), jnp.float32),
            pltpu.VMEM((_RPW,), jnp.int32),
        ],
    )


def kernel(x, enc_w1, enc_b1, enc_w2, enc_b2, kp_w1, kp_b1, kp_w2, kp_b2,
           kp_w3, kp_b3, k_scale):
    logits, kvec, nvec = _tc_call(
        x, enc_w1, enc_b1, enc_w2, enc_b2, kp_w1, kp_b1, kp_w2, kp_b2,
        kp_w3.reshape(1, N_HDIM_), kp_b3.reshape(1, 1), k_scale.reshape(1, 1))
    khot = _sc_mask_call()(logits, nvec)
    return khot, kvec.reshape(BATCH_, 1)


# speedup vs baseline: 5.1138x; 2.5433x over previous
import functools

import jax
import jax.numpy as jnp
import numpy as np
from jax import lax
from jax.experimental import pallas as pl
from jax.experimental.pallas import tpu as pltpu
from jax.experimental.pallas import tpu_sc as plsc

INPUT_DIM_ = 1024
N_HDIM_ = 512
QDIM_ = 4096
BATCH_ = 4096

_BM = 256
_GRID = BATCH_ // _BM
_PREC = lax.Precision.DEFAULT

_NC = 2
_NS = 16
_NW = _NC * _NS
_RPW = BATCH_ // _NW
_L = 16
_NCH = QDIM_ // _L
_SIGN = np.int32(-(2**31))


def _dot(a, b):
    return lax.dot_general(a, b, (((1,), (0,)), ((), ())),
                           precision=_PREC,
                           preferred_element_type=jnp.float32)


def _tc_body(x_ref, w1_ref, b1_ref, w2_ref, b2_ref, kpw1_ref, kpb1_ref,
             kpw2_ref, kpb2_ref, kpw3_ref, kpb3_ref, ksc_ref,
             logits_ref, k_ref, n_ref):
    x = x_ref[...]
    h = jnp.maximum(_dot(x, w1_ref[...]) + b1_ref[...][None, :], 0.0)
    logits = _dot(h, w2_ref[...]) + b2_ref[...][None, :]
    logits_ref[...] = logits
    h1 = jnp.maximum(_dot(x, kpw1_ref[0:INPUT_DIM_, :])
                     + _dot(logits, kpw1_ref[INPUT_DIM_:, :])
                     + kpb1_ref[...][None, :], 0.0)
    h2 = jnp.maximum(_dot(h1, kpw2_ref[...]) + kpb2_ref[...][None, :], 0.0)
    z = jnp.sum(h2 * kpw3_ref[...], axis=-1, keepdims=True) + kpb3_ref[...]
    k = jax.nn.sigmoid(z) * float(QDIM_)
    k = jnp.clip(k * jax.nn.sigmoid(ksc_ref[...]) * 2.0, 1.0, float(QDIM_))
    k_ref[...] = k[:, 0]
    n_ref[...] = jnp.ceil(k[:, 0]).astype(jnp.int32)


def _tc_call(x, enc_w1, enc_b1, enc_w2, enc_b2, kp_w1, kp_b1, kp_w2, kp_b2,
             kp_w3_row, kp_b3s, k_scales):
    return pl.pallas_call(
        _tc_body,
        grid=(_GRID,),
        in_specs=[
            pl.BlockSpec((_BM, INPUT_DIM_), lambda i: (i, 0)),
            pl.BlockSpec((INPUT_DIM_, N_HDIM_), lambda i: (0, 0)),
            pl.BlockSpec((N_HDIM_,), lambda i: (0,)),
            pl.BlockSpec((N_HDIM_, QDIM_), lambda i: (0, 0)),
            pl.BlockSpec((QDIM_,), lambda i: (0,)),
            pl.BlockSpec((INPUT_DIM_ + QDIM_, N_HDIM_), lambda i: (0, 0)),
            pl.BlockSpec((N_HDIM_,), lambda i: (0,)),
            pl.BlockSpec((N_HDIM_, N_HDIM_), lambda i: (0, 0)),
            pl.BlockSpec((N_HDIM_,), lambda i: (0,)),
            pl.BlockSpec((1, N_HDIM_), lambda i: (0, 0)),
            pl.BlockSpec((1, 1), lambda i: (0, 0)),
            pl.BlockSpec((1, 1), lambda i: (0, 0)),
        ],
        out_specs=[
            pl.BlockSpec((_BM, QDIM_), lambda i: (i, 0)),
            pl.BlockSpec((_BM,), lambda i: (i,)),
            pl.BlockSpec((_BM,), lambda i: (i,)),
        ],
        out_shape=[
            jax.ShapeDtypeStruct((BATCH_, QDIM_), jnp.float32),
            jax.ShapeDtypeStruct((BATCH_,), jnp.float32),
            jax.ShapeDtypeStruct((BATCH_,), jnp.int32),
        ],
    )(x, enc_w1, enc_b1, enc_w2, enc_b2, kp_w1, kp_b1, kp_w2, kp_b2,
      kp_w3_row, kp_b3s, k_scales)


def _sc_body(logits_hbm, n_hbm, out_hbm, row_v, s_v, out_v, n_v,
             hist_v, totals_v, cand_v, cand2_v):
    cid = lax.axis_index("c")
    sid = lax.axis_index("s")
    wid = sid * _NC + cid
    base = wid * _RPW
    pltpu.sync_copy(n_hbm.at[pl.ds(base, _RPW)], n_v)

    lane = lax.broadcasted_iota(jnp.int32, (_L,), 0)
    laneoff = lane << 8
    zero_v = jnp.zeros((_L,), jnp.int32)
    ones_v = jnp.full((_L,), 1, jnp.int32)
    pad_v = jnp.full((_L,), _SIGN, jnp.int32)

    def zero_hist():
        def zc(ci, c):
            hist_v[pl.ds(ci * _L, _L)] = zero_v
            return c
        lax.fori_loop(0, 256, zc, 0)

    def merge_hist():
        def mg(j, c):
            tot = hist_v[pl.ds(j * _L, _L)]
            for reg in range(1, 16):
                tot = tot + hist_v[pl.ds(reg * 256 + j * _L, _L)]
            totals_v[pl.ds(j * _L, _L)] = tot
            return c
        lax.fori_loop(0, 16, mg, 0)

    def find_boundary(rem):
        def fd(j, st):
            carry, dcount = st
            tot = totals_v[pl.ds(j * _L, _L)]
            incl = plsc.cumsum(tot)
            excl = incl - tot
            cond = (excl + carry) <= rem
            dcount = dcount + plsc.all_reduce_population_count(cond)[0]
            carry = carry + incl[15]
            return carry, dcount
        _, dcount = lax.fori_loop(0, 16, fd, (jnp.int32(0), jnp.int32(0)))
        D = dcount - 1

        def acc2(j, st):
            accA, accC = st
            tot = totals_v[pl.ds(j * _L, _L)]
            dig = lane + j * _L
            accA = accA + jnp.where(dig <= D, tot, 0)
            accC = accC + jnp.where(dig == D, tot, 0)
            return accA, accC
        accA, accC = lax.fori_loop(0, 16, acc2, (zero_v, zero_v))
        return D, jnp.sum(accA), jnp.sum(accC)

    def do_row(r, carry):
        row = base + r
        pltpu.sync_copy(logits_hbm.at[row], row_v)
        nchunk = n_v[pl.ds((r // _L) * _L, _L)]
        n = jnp.sum(jnp.where(lane == r % _L, nchunk, 0))

        zero_hist()

        def maphist(ci, c):
            f = row_v[pl.ds(ci * _L, _L)]
            b = plsc.bitcast(f, jnp.int32)
            sv = jnp.where(b < 0, -(b & jnp.int32(0x7FFFFFFF)), b)
            s_v[pl.ds(ci * _L, _L)] = sv
            d = lax.shift_right_logical(sv, 24) ^ 0x80
            plsc.addupdate_scatter(hist_v, [laneoff | d], ones_v)
            return c
        lax.fori_loop(0, _NCH, maphist, 0)

        merge_hist()
        D, pinc, c_cand = find_boundary(QDIM_ - n)
        c_gt_b = QDIM_ - pinc
        np1 = n - c_gt_b

        def compact(ci, off):
            sv = s_v[pl.ds(ci * _L, _L)]
            d = lax.shift_right_logical(sv, 24) ^ 0x80
            m = d == D
            plsc.store_compressed(cand_v.at[pl.ds(off, _L)], sv, mask=m)
            return off + plsc.all_reduce_population_count(m)[0]
        off = lax.fori_loop(0, _NCH, compact, jnp.int32(0))
        cand_v[pl.ds(off, _L)] = pad_v
        nch2 = (c_cand + (_L - 1)) // _L
        ntot2 = nch2 * _L

        zero_hist()

        def hist2(ci, c):
            sv = cand_v[pl.ds(ci * _L, _L)]
            d2 = lax.shift_right_logical(sv, 16) & 0xFF
            plsc.addupdate_scatter(hist_v, [laneoff | d2], ones_v)
            return c
        lax.fori_loop(0, nch2, hist2, 0)

        merge_hist()
        D2, pinc2, _ = find_boundary(ntot2 - np1)
        c_gt_b2 = ntot2 - pinc2
        np2 = np1 - c_gt_b2

        def compact2(ci, off2):
            sv = cand_v[pl.ds(ci * _L, _L)]
            d2 = lax.shift_right_logical(sv, 16) & 0xFF
            m = d2 == D2
            plsc.store_compressed(cand2_v.at[pl.ds(off2, _L)], sv, mask=m)
            return off2 + plsc.all_reduce_population_count(m)[0]
        off2 = lax.fori_loop(0, nch2, compact2, jnp.int32(0))
        cand2_v[pl.ds(off2, _L)] = pad_v
        nch3 = (off2 + (_L - 1)) // _L

        t0 = (D << 24) | (D2 << 16)

        def bit_step(i, t_u):
            bit = 15 - i
            t2 = t_u | (jnp.int32(1) << bit)
            thr_s = t2 ^ _SIGN

            def cc(ci, acc):
                sv = cand2_v[pl.ds(ci * _L, _L)]
                return acc + jnp.where(sv >= thr_s, 1, 0)
            cnt = jnp.sum(lax.fori_loop(0, nch3, cc, zero_v))
            return jnp.where(cnt >= np2, t2, t_u)
        t_u = lax.fori_loop(0, 16, bit_step, t0)
        thr = t_u ^ _SIGN

        def cgt_in(ci, acc):
            sv = cand2_v[pl.ds(ci * _L, _L)]
            return acc + jnp.where(sv > thr, 1, 0)
        c_gt = c_gt_b + c_gt_b2 + jnp.sum(
            lax.fori_loop(0, nch3, cgt_in, zero_v))
        rneed = n - c_gt

        def emit_chunk(ci, seen):
            sv = s_v[pl.ds(ci * _L, _L)]
            m_gt = sv > thr
            m_eq = sv == thr
            eq = jnp.where(m_eq, 1, 0).astype(jnp.int32)
            pref = plsc.cumsum(eq)
            take = m_gt | (m_eq & ((seen + pref) <= rneed))
            out_v[pl.ds(ci * _L, _L)] = jnp.where(take, 1.0, 0.0).astype(jnp.float32)
            return seen + jnp.sum(eq)
        lax.fori_loop(0, _NCH, emit_chunk, jnp.int32(0))
        pltpu.sync_copy(out_v, out_hbm.at[row])
        return carry
    lax.fori_loop(0, _RPW, do_row, 0)


@functools.cache
def _sc_mask_call():
    return pl.kernel(
        _sc_body,
        out_type=jax.ShapeDtypeStruct((BATCH_, QDIM_), jnp.float32),
        mesh=plsc.VectorSubcoreMesh(core_axis_name="c", subcore_axis_name="s",
                                    num_cores=_NC, num_subcores=_NS),
        compiler_params=pltpu.CompilerParams(needs_layout_passes=False),
        scratch_types=[
            pltpu.VMEM((QDIM_,), jnp.float32),
            pltpu.VMEM((QDIM_,), jnp.int32),
            pltpu.VMEM((QDIM_,), jnp.float32),
            pltpu.VMEM((_RPW,), jnp.int32),
            pltpu.VMEM((4096,), jnp.int32),
            pltpu.VMEM((256,), jnp.int32),
            pltpu.VMEM((QDIM_ + _L,), jnp.int32),
            pltpu.VMEM((QDIM_ + _L,), jnp.int32),
        ],
    )


def kernel(x, enc_w1, enc_b1, enc_w2, enc_b2, kp_w1, kp_b1, kp_w2, kp_b2,
           kp_w3, kp_b3, k_scale):
    logits, kvec, nvec = _tc_call(
        x, enc_w1, enc_b1, enc_w2, enc_b2, kp_w1, kp_b1, kp_w2, kp_b2,
        kp_w3.reshape(1, N_HDIM_), kp_b3.reshape(1, 1), k_scale.reshape(1, 1))
    khot = _sc_mask_call()(logits, nvec)
    return khot, kvec.reshape(BATCH_, 1)


# speedup vs baseline: 6.5147x; 1.2739x over previous
import functools

import jax
import jax.numpy as jnp
import numpy as np
from jax import lax
from jax.experimental import pallas as pl
from jax.experimental.pallas import tpu as pltpu
from jax.experimental.pallas import tpu_sc as plsc

INPUT_DIM_ = 1024
N_HDIM_ = 512
QDIM_ = 4096
BATCH_ = 4096

_BM = 256
_GRID = BATCH_ // _BM
_PREC = lax.Precision.DEFAULT

_NC = 2
_NS = 16
_NW = _NC * _NS
_RPW = BATCH_ // _NW
_L = 16
_NCH = QDIM_ // _L
_SIGN = np.int32(-(2**31))


def _dot(a, b):
    return lax.dot_general(a, b, (((1,), (0,)), ((), ())),
                           precision=_PREC,
                           preferred_element_type=jnp.float32)


def _tc_body(x_ref, w1_ref, b1_ref, w2_ref, b2_ref, kpw1_ref, kpb1_ref,
             kpw2_ref, kpb2_ref, kpw3_ref, kpb3_ref, ksc_ref,
             logits_ref, k_ref, n_ref):
    x = x_ref[...]
    h = jnp.maximum(_dot(x, w1_ref[...]) + b1_ref[...][None, :], 0.0)
    logits = _dot(h, w2_ref[...]) + b2_ref[...][None, :]
    logits_ref[...] = logits
    h1 = jnp.maximum(_dot(x, kpw1_ref[0:INPUT_DIM_, :])
                     + _dot(logits, kpw1_ref[INPUT_DIM_:, :])
                     + kpb1_ref[...][None, :], 0.0)
    h2 = jnp.maximum(_dot(h1, kpw2_ref[...]) + kpb2_ref[...][None, :], 0.0)
    z = jnp.sum(h2 * kpw3_ref[...], axis=-1, keepdims=True) + kpb3_ref[...]
    k = jax.nn.sigmoid(z) * float(QDIM_)
    k = jnp.clip(k * jax.nn.sigmoid(ksc_ref[...]) * 2.0, 1.0, float(QDIM_))
    k_ref[...] = k[:, 0]
    n_ref[...] = jnp.ceil(k[:, 0]).astype(jnp.int32)


def _tc_call(x, enc_w1, enc_b1, enc_w2, enc_b2, kp_w1, kp_b1, kp_w2, kp_b2,
             kp_w3_row, kp_b3s, k_scales):
    return pl.pallas_call(
        _tc_body,
        grid=(_GRID,),
        in_specs=[
            pl.BlockSpec((_BM, INPUT_DIM_), lambda i: (i, 0)),
            pl.BlockSpec((INPUT_DIM_, N_HDIM_), lambda i: (0, 0)),
            pl.BlockSpec((N_HDIM_,), lambda i: (0,)),
            pl.BlockSpec((N_HDIM_, QDIM_), lambda i: (0, 0)),
            pl.BlockSpec((QDIM_,), lambda i: (0,)),
            pl.BlockSpec((INPUT_DIM_ + QDIM_, N_HDIM_), lambda i: (0, 0)),
            pl.BlockSpec((N_HDIM_,), lambda i: (0,)),
            pl.BlockSpec((N_HDIM_, N_HDIM_), lambda i: (0, 0)),
            pl.BlockSpec((N_HDIM_,), lambda i: (0,)),
            pl.BlockSpec((1, N_HDIM_), lambda i: (0, 0)),
            pl.BlockSpec((1, 1), lambda i: (0, 0)),
            pl.BlockSpec((1, 1), lambda i: (0, 0)),
        ],
        out_specs=[
            pl.BlockSpec((_BM, QDIM_), lambda i: (i, 0)),
            pl.BlockSpec((_BM,), lambda i: (i,)),
            pl.BlockSpec((_BM,), lambda i: (i,)),
        ],
        out_shape=[
            jax.ShapeDtypeStruct((BATCH_, QDIM_), jnp.float32),
            jax.ShapeDtypeStruct((BATCH_,), jnp.float32),
            jax.ShapeDtypeStruct((BATCH_,), jnp.int32),
        ],
    )(x, enc_w1, enc_b1, enc_w2, enc_b2, kp_w1, kp_b1, kp_w2, kp_b2,
      kp_w3_row, kp_b3s, k_scales)


def _sc_body(logits_hbm, n_hbm, out_hbm, row_v, s_v, out_v, n_v,
             hist_v, totals_v, cand_v, cand2_v, sem_in, sem_out):
    cid = lax.axis_index("c")
    sid = lax.axis_index("s")
    wid = sid * _NC + cid
    base = wid * _RPW
    pltpu.sync_copy(n_hbm.at[pl.ds(base, _RPW)], n_v)
    pltpu.async_copy(logits_hbm.at[base], row_v.at[pl.ds(0, QDIM_)], sem_in)

    lane = lax.broadcasted_iota(jnp.int32, (_L,), 0)
    laneoff = lane << 8
    zero_v = jnp.zeros((_L,), jnp.int32)
    ones_v = jnp.full((_L,), 1, jnp.int32)
    pad_v = jnp.full((_L,), _SIGN, jnp.int32)

    def zero_hist():
        def zc(ci, c):
            hist_v[pl.ds(ci * _L, _L)] = zero_v
            return c
        lax.fori_loop(0, 256, zc, 0, unroll=8)

    def merge_hist():
        def mg(j, c):
            tot = hist_v[pl.ds(j * _L, _L)]
            for reg in range(1, 16):
                tot = tot + hist_v[pl.ds(reg * 256 + j * _L, _L)]
            totals_v[pl.ds(j * _L, _L)] = tot
            return c
        lax.fori_loop(0, 16, mg, 0)

    def find_boundary(rem):
        def fd(j, st):
            carry, dcount = st
            tot = totals_v[pl.ds(j * _L, _L)]
            incl = plsc.cumsum(tot)
            excl = incl - tot
            cond = (excl + carry) <= rem
            dcount = dcount + plsc.all_reduce_population_count(cond)[0]
            carry = carry + incl[15]
            return carry, dcount
        _, dcount = lax.fori_loop(0, 16, fd, (jnp.int32(0), jnp.int32(0)))
        D = dcount - 1

        def acc2(j, st):
            accA, accC = st
            tot = totals_v[pl.ds(j * _L, _L)]
            dig = lane + j * _L
            accA = accA + jnp.where(dig <= D, tot, 0)
            accC = accC + jnp.where(dig == D, tot, 0)
            return accA, accC
        accA, accC = lax.fori_loop(0, 16, acc2, (zero_v, zero_v))
        return D, jnp.sum(accA), jnp.sum(accC)

    def do_row(r, carry):
        row = base + r
        ro = (r % 2) * QDIM_
        oo = (r % 2) * QDIM_
        rb = row_v.at[pl.ds(ro, QDIM_)]
        ob = out_v.at[pl.ds(oo, QDIM_)]
        pltpu.make_async_copy(logits_hbm.at[row], rb, sem_in).wait()

        @pl.when(r + 1 < _RPW)
        def _():
            pltpu.async_copy(logits_hbm.at[row + 1],
                             row_v.at[pl.ds(((r + 1) % 2) * QDIM_, QDIM_)],
                             sem_in)
        nchunk = n_v[pl.ds((r // _L) * _L, _L)]
        n = jnp.sum(jnp.where(lane == r % _L, nchunk, 0))

        zero_hist()

        def maphist(ci, c):
            f = rb[pl.ds(ci * _L, _L)]
            b = plsc.bitcast(f, jnp.int32)
            sv = jnp.where(b < 0, -(b & jnp.int32(0x7FFFFFFF)), b)
            s_v[pl.ds(ci * _L, _L)] = sv
            d = lax.shift_right_logical(sv, 24) ^ 0x80
            plsc.addupdate_scatter(hist_v, [laneoff | d], ones_v)
            return c
        lax.fori_loop(0, _NCH, maphist, 0, unroll=4)

        merge_hist()
        D, pinc, c_cand = find_boundary(QDIM_ - n)
        c_gt_b = QDIM_ - pinc
        np1 = n - c_gt_b

        def compact(ci, off):
            sv = s_v[pl.ds(ci * _L, _L)]
            d = lax.shift_right_logical(sv, 24) ^ 0x80
            m = d == D
            plsc.store_compressed(cand_v.at[pl.ds(off, _L)], sv, mask=m)
            return off + plsc.all_reduce_population_count(m)[0]
        off = lax.fori_loop(0, _NCH, compact, jnp.int32(0), unroll=2)
        cand_v[pl.ds(off, _L)] = pad_v
        nch2 = (c_cand + (_L - 1)) // _L
        ntot2 = nch2 * _L

        zero_hist()

        def hist2(ci, c):
            sv = cand_v[pl.ds(ci * _L, _L)]
            d2 = lax.shift_right_logical(sv, 16) & 0xFF
            plsc.addupdate_scatter(hist_v, [laneoff | d2], ones_v)
            return c
        lax.fori_loop(0, nch2, hist2, 0)

        merge_hist()
        D2, pinc2, _ = find_boundary(ntot2 - np1)
        c_gt_b2 = ntot2 - pinc2
        np2 = np1 - c_gt_b2

        def compact2(ci, off2):
            sv = cand_v[pl.ds(ci * _L, _L)]
            d2 = lax.shift_right_logical(sv, 16) & 0xFF
            m = d2 == D2
            plsc.store_compressed(cand2_v.at[pl.ds(off2, _L)], sv, mask=m)
            return off2 + plsc.all_reduce_population_count(m)[0]
        off2 = lax.fori_loop(0, nch2, compact2, jnp.int32(0))
        cand2_v[pl.ds(off2, _L)] = pad_v
        nch3 = (off2 + (_L - 1)) // _L

        t0 = (D << 24) | (D2 << 16)

        def bit_step(i, t_u):
            bit = 15 - i
            t2 = t_u | (jnp.int32(1) << bit)
            thr_s = t2 ^ _SIGN

            def cc(ci, acc):
                sv = cand2_v[pl.ds(ci * _L, _L)]
                return acc + jnp.where(sv >= thr_s, 1, 0)
            cnt = jnp.sum(lax.fori_loop(0, nch3, cc, zero_v))
            return jnp.where(cnt >= np2, t2, t_u)
        t_u = lax.fori_loop(0, 16, bit_step, t0)
        thr = t_u ^ _SIGN

        def cnt_in(ci, st):
            g, e = st
            sv = cand2_v[pl.ds(ci * _L, _L)]
            g = g + jnp.where(sv > thr, 1, 0)
            e = e + jnp.where(sv == thr, 1, 0)
            return g, e
        gacc, eacc = lax.fori_loop(0, nch3, cnt_in, (zero_v, zero_v))
        c_gt = c_gt_b + c_gt_b2 + jnp.sum(gacc)
        c_eq = jnp.sum(eacc)
        rneed = n - c_gt

        @pl.when(r >= 1)
        def _():
            pltpu.make_async_copy(
                out_v.at[pl.ds(((r - 1) % 2) * QDIM_, QDIM_)],
                out_hbm.at[row - 1], sem_out).wait()

        def emit_chunk(ci, c):
            sv = s_v[pl.ds(ci * _L, _L)]
            ob[pl.ds(ci * _L, _L)] = jnp.where(sv >= thr, 1.0, 0.0)
            return c
        lax.fori_loop(0, _NCH, emit_chunk, 0, unroll=8)

        def fix_cond(st):
            return st[1] > 0

        def fix_body(st):
            ci, extra = st
            sv = s_v[pl.ds(ci * _L, _L)]
            m_eq = sv == thr
            eq = jnp.where(m_eq, 1, 0).astype(jnp.int32)
            ec = plsc.all_reduce_population_count(m_eq)[0]
            pref = plsc.cumsum(eq)
            clear = m_eq & (pref > (ec - extra))
            cur = ob[pl.ds(ci * _L, _L)]
            ob[pl.ds(ci * _L, _L)] = jnp.where(clear, 0.0, cur)
            return ci - 1, extra - jnp.minimum(ec, extra)
        lax.while_loop(fix_cond, fix_body,
                       (jnp.int32(_NCH - 1), c_eq - rneed))

        pltpu.async_copy(ob, out_hbm.at[row], sem_out)
        return carry
    lax.fori_loop(0, _RPW, do_row, 0)
    pltpu.make_async_copy(out_v.at[pl.ds(((_RPW - 1) % 2) * QDIM_, QDIM_)],
                          out_hbm.at[base + _RPW - 1], sem_out).wait()


@functools.cache
def _sc_mask_call():
    return pl.kernel(
        _sc_body,
        out_type=jax.ShapeDtypeStruct((BATCH_, QDIM_), jnp.float32),
        mesh=plsc.VectorSubcoreMesh(core_axis_name="c", subcore_axis_name="s",
                                    num_cores=_NC, num_subcores=_NS),
        compiler_params=pltpu.CompilerParams(needs_layout_passes=False),
        scratch_types=[
            pltpu.VMEM((2 * QDIM_,), jnp.float32),
            pltpu.VMEM((QDIM_,), jnp.int32),
            pltpu.VMEM((2 * QDIM_,), jnp.float32),
            pltpu.VMEM((_RPW,), jnp.int32),
            pltpu.VMEM((4096,), jnp.int32),
            pltpu.VMEM((256,), jnp.int32),
            pltpu.VMEM((QDIM_ + _L,), jnp.int32),
            pltpu.VMEM((QDIM_ + _L,), jnp.int32),
            pltpu.SemaphoreType.DMA,
            pltpu.SemaphoreType.DMA,
        ],
    )


def kernel(x, enc_w1, enc_b1, enc_w2, enc_b2, kp_w1, kp_b1, kp_w2, kp_b2,
           kp_w3, kp_b3, k_scale):
    logits, kvec, nvec = _tc_call(
        x, enc_w1, enc_b1, enc_w2, enc_b2, kp_w1, kp_b1, kp_w2, kp_b2,
        kp_w3.reshape(1, N_HDIM_), kp_b3.reshape(1, 1), k_scale.reshape(1, 1))
    khot = _sc_mask_call()(logits, nvec)
    return khot, kvec.reshape(BATCH_, 1)


# speedup vs baseline: 16.8773x; 2.5906x over previous
import functools

import jax
import jax.numpy as jnp
import numpy as np
from jax import lax
from jax.experimental import pallas as pl
from jax.experimental.pallas import tpu as pltpu
from jax.experimental.pallas import tpu_sc as plsc

INPUT_DIM_ = 1024
N_HDIM_ = 512
QDIM_ = 4096
BATCH_ = 4096

_BM = 256
_GRID = BATCH_ // _BM
_PREC = lax.Precision.DEFAULT

_NC = 2
_NS = 16
_NW = _NC * _NS
_RPW = BATCH_ // _NW
_L = 16
_NCH = QDIM_ // _L
_SIGN = np.int32(-(2**31))


def _dot(a, b):
    return lax.dot_general(a, b, (((1,), (0,)), ((), ())),
                           precision=_PREC,
                           preferred_element_type=jnp.float32)


def _tc_body(x_ref, w1_ref, b1_ref, w2_ref, b2_ref, kpw1_ref, kpb1_ref,
             kpw2_ref, kpb2_ref, kpw3_ref, kpb3_ref, ksc_ref,
             logits_ref, k_ref, n_ref):
    x = x_ref[...]
    h = jnp.maximum(_dot(x, w1_ref[...]) + b1_ref[...][None, :], 0.0)
    logits = _dot(h, w2_ref[...]) + b2_ref[...][None, :]
    logits_ref[...] = logits
    h1 = jnp.maximum(_dot(x, kpw1_ref[0:INPUT_DIM_, :])
                     + _dot(logits, kpw1_ref[INPUT_DIM_:, :])
                     + kpb1_ref[...][None, :], 0.0)
    h2 = jnp.maximum(_dot(h1, kpw2_ref[...]) + kpb2_ref[...][None, :], 0.0)
    z = jnp.sum(h2 * kpw3_ref[...], axis=-1, keepdims=True) + kpb3_ref[...]
    k = jax.nn.sigmoid(z) * float(QDIM_)
    k = jnp.clip(k * jax.nn.sigmoid(ksc_ref[...]) * 2.0, 1.0, float(QDIM_))
    k_ref[...] = k[:, 0]
    n_ref[...] = jnp.ceil(k[:, 0]).astype(jnp.int32)


def _tc_call(x, enc_w1, enc_b1, enc_w2, enc_b2, kp_w1, kp_b1, kp_w2, kp_b2,
             kp_w3_row, kp_b3s, k_scales):
    return pl.pallas_call(
        _tc_body,
        grid=(_GRID,),
        in_specs=[
            pl.BlockSpec((_BM, INPUT_DIM_), lambda i: (i, 0)),
            pl.BlockSpec((INPUT_DIM_, N_HDIM_), lambda i: (0, 0)),
            pl.BlockSpec((N_HDIM_,), lambda i: (0,)),
            pl.BlockSpec((N_HDIM_, QDIM_), lambda i: (0, 0)),
            pl.BlockSpec((QDIM_,), lambda i: (0,)),
            pl.BlockSpec((INPUT_DIM_ + QDIM_, N_HDIM_), lambda i: (0, 0)),
            pl.BlockSpec((N_HDIM_,), lambda i: (0,)),
            pl.BlockSpec((N_HDIM_, N_HDIM_), lambda i: (0, 0)),
            pl.BlockSpec((N_HDIM_,), lambda i: (0,)),
            pl.BlockSpec((1, N_HDIM_), lambda i: (0, 0)),
            pl.BlockSpec((1, 1), lambda i: (0, 0)),
            pl.BlockSpec((1, 1), lambda i: (0, 0)),
        ],
        out_specs=[
            pl.BlockSpec((_BM, QDIM_), lambda i: (i, 0)),
            pl.BlockSpec((_BM,), lambda i: (i,)),
            pl.BlockSpec((_BM,), lambda i: (i,)),
        ],
        out_shape=[
            jax.ShapeDtypeStruct((BATCH_, QDIM_), jnp.float32),
            jax.ShapeDtypeStruct((BATCH_,), jnp.float32),
            jax.ShapeDtypeStruct((BATCH_,), jnp.int32),
        ],
    )(x, enc_w1, enc_b1, enc_w2, enc_b2, kp_w1, kp_b1, kp_w2, kp_b2,
      kp_w3_row, kp_b3s, k_scales)


def _sc_body(logits_hbm, n_hbm, out_hbm, row_v, s_v, out_v, n_v,
             hist_v, totals_v, cand_v, cand2_v, sem_in, sem_out):
    cid = lax.axis_index("c")
    sid = lax.axis_index("s")
    wid = sid * _NC + cid
    base = wid * _RPW
    pltpu.sync_copy(n_hbm.at[pl.ds(base, _RPW)], n_v)
    pltpu.async_copy(logits_hbm.at[base], row_v.at[pl.ds(0, QDIM_)], sem_in)

    lane = lax.broadcasted_iota(jnp.int32, (_L,), 0)
    laneoff = lane << 8
    zero_v = jnp.zeros((_L,), jnp.int32)
    ones_v = jnp.full((_L,), 1, jnp.int32)
    pad_v = jnp.full((_L,), _SIGN, jnp.int32)

    def zero_hist():
        @plsc.parallel_loop(0, 256, unroll=8)
        def _(ci):
            hist_v[pl.ds(ci * _L, _L)] = zero_v

    def merge_hist():
        def mg(j, c):
            tot = hist_v[pl.ds(j * _L, _L)]
            for reg in range(1, 16):
                tot = tot + hist_v[pl.ds(reg * 256 + j * _L, _L)]
            totals_v[pl.ds(j * _L, _L)] = tot
            return c
        lax.fori_loop(0, 16, mg, 0)

    def find_boundary(rem):
        def fd(j, st):
            carry, dcount = st
            tot = totals_v[pl.ds(j * _L, _L)]
            incl = plsc.cumsum(tot)
            excl = incl - tot
            cond = (excl + carry) <= rem
            dcount = dcount + plsc.all_reduce_population_count(cond)[0]
            carry = carry + incl[15]
            return carry, dcount
        _, dcount = lax.fori_loop(0, 16, fd, (jnp.int32(0), jnp.int32(0)))
        D = dcount - 1

        def acc2(j, st):
            accA, accC = st
            tot = totals_v[pl.ds(j * _L, _L)]
            dig = lane + j * _L
            accA = accA + jnp.where(dig <= D, tot, 0)
            accC = accC + jnp.where(dig == D, tot, 0)
            return accA, accC
        accA, accC = lax.fori_loop(0, 16, acc2, (zero_v, zero_v))
        return D, jnp.sum(accA), jnp.sum(accC)

    def do_row(r, carry):
        row = base + r
        ro = (r % 2) * QDIM_
        oo = (r % 2) * QDIM_
        rb = row_v.at[pl.ds(ro, QDIM_)]
        ob = out_v.at[pl.ds(oo, QDIM_)]
        pltpu.make_async_copy(logits_hbm.at[row], rb, sem_in).wait()

        @pl.when(r + 1 < _RPW)
        def _():
            pltpu.async_copy(logits_hbm.at[row + 1],
                             row_v.at[pl.ds(((r + 1) % 2) * QDIM_, QDIM_)],
                             sem_in)
        nchunk = n_v[pl.ds((r // _L) * _L, _L)]
        n = jnp.sum(jnp.where(lane == r % _L, nchunk, 0))

        zero_hist()

        @plsc.parallel_loop(0, _NCH, unroll=4)
        def _(ci):
            f = rb[pl.ds(ci * _L, _L)]
            b = plsc.bitcast(f, jnp.int32)
            sv = jnp.where(b < 0, -(b & jnp.int32(0x7FFFFFFF)), b)
            s_v[pl.ds(ci * _L, _L)] = sv
            d = lax.shift_right_logical(sv, 24) ^ 0x80
            plsc.addupdate_scatter(hist_v, [laneoff | d], ones_v)

        merge_hist()
        D, pinc, c_cand = find_boundary(QDIM_ - n)
        c_gt_b = QDIM_ - pinc
        np1 = n - c_gt_b

        @plsc.parallel_loop(0, _NCH, unroll=4, carry=jnp.int32(0))
        def off(ci, off_c):
            sv = s_v[pl.ds(ci * _L, _L)]
            d = lax.shift_right_logical(sv, 24) ^ 0x80
            m = d == D
            plsc.store_compressed(cand_v.at[pl.ds(off_c, _L)], sv, mask=m)
            return off_c + plsc.all_reduce_population_count(m)[0]
        cand_v[pl.ds(off, _L)] = pad_v
        nch2 = (c_cand + (_L - 1)) // _L
        ntot2 = nch2 * _L

        zero_hist()

        @plsc.parallel_loop(0, nch2, unroll=2)
        def _(ci):
            sv = cand_v[pl.ds(ci * _L, _L)]
            d2 = lax.shift_right_logical(sv, 16) & 0xFF
            plsc.addupdate_scatter(hist_v, [laneoff | d2], ones_v)

        merge_hist()
        D2, pinc2, _ = find_boundary(ntot2 - np1)
        c_gt_b2 = ntot2 - pinc2
        np2 = np1 - c_gt_b2

        @plsc.parallel_loop(0, nch2, unroll=2, carry=jnp.int32(0))
        def off2(ci, off_c):
            sv = cand_v[pl.ds(ci * _L, _L)]
            d2 = lax.shift_right_logical(sv, 16) & 0xFF
            m = d2 == D2
            plsc.store_compressed(cand2_v.at[pl.ds(off_c, _L)], sv, mask=m)
            return off_c + plsc.all_reduce_population_count(m)[0]
        cand2_v[pl.ds(off2, _L)] = pad_v
        nch3 = (off2 + (_L - 1)) // _L

        t0 = (D << 24) | (D2 << 16)

        def bit_step(i, t_u):
            bit = 15 - i
            t2 = t_u | (jnp.int32(1) << bit)
            thr_s = t2 ^ _SIGN

            def cc(ci, acc):
                sv = cand2_v[pl.ds(ci * _L, _L)]
                return acc + jnp.where(sv >= thr_s, 1, 0)
            cnt = jnp.sum(lax.fori_loop(0, nch3, cc, zero_v))
            return jnp.where(cnt >= np2, t2, t_u)
        t_u = lax.fori_loop(0, 16, bit_step, t0)
        thr = t_u ^ _SIGN

        def cnt_in(ci, st):
            g, e = st
            sv = cand2_v[pl.ds(ci * _L, _L)]
            g = g + jnp.where(sv > thr, 1, 0)
            e = e + jnp.where(sv == thr, 1, 0)
            return g, e
        gacc, eacc = lax.fori_loop(0, nch3, cnt_in, (zero_v, zero_v))
        c_gt = c_gt_b + c_gt_b2 + jnp.sum(gacc)
        c_eq = jnp.sum(eacc)
        rneed = n - c_gt

        @pl.when(r >= 1)
        def _():
            pltpu.make_async_copy(
                out_v.at[pl.ds(((r - 1) % 2) * QDIM_, QDIM_)],
                out_hbm.at[row - 1], sem_out).wait()

        @plsc.parallel_loop(0, _NCH, unroll=8)
        def _(ci):
            sv = s_v[pl.ds(ci * _L, _L)]
            ob[pl.ds(ci * _L, _L)] = jnp.where(sv >= thr, 1.0, 0.0)

        def fix_cond(st):
            return st[1] > 0

        def fix_body(st):
            ci, extra = st
            sv = s_v[pl.ds(ci * _L, _L)]
            m_eq = sv == thr
            eq = jnp.where(m_eq, 1, 0).astype(jnp.int32)
            ec = plsc.all_reduce_population_count(m_eq)[0]
            pref = plsc.cumsum(eq)
            clear = m_eq & (pref > (ec - extra))
            cur = ob[pl.ds(ci * _L, _L)]
            ob[pl.ds(ci * _L, _L)] = jnp.where(clear, 0.0, cur)
            return ci - 1, extra - jnp.minimum(ec, extra)
        lax.while_loop(fix_cond, fix_body,
                       (jnp.int32(_NCH - 1), c_eq - rneed))

        pltpu.async_copy(ob, out_hbm.at[row], sem_out)
        return carry
    lax.fori_loop(0, _RPW, do_row, 0)
    pltpu.make_async_copy(out_v.at[pl.ds(((_RPW - 1) % 2) * QDIM_, QDIM_)],
                          out_hbm.at[base + _RPW - 1], sem_out).wait()


@functools.cache
def _sc_mask_call():
    return pl.kernel(
        _sc_body,
        out_type=jax.ShapeDtypeStruct((BATCH_, QDIM_), jnp.float32),
        mesh=plsc.VectorSubcoreMesh(core_axis_name="c", subcore_axis_name="s",
                                    num_cores=_NC, num_subcores=_NS),
        compiler_params=pltpu.CompilerParams(needs_layout_passes=False),
        scratch_types=[
            pltpu.VMEM((2 * QDIM_,), jnp.float32),
            pltpu.VMEM((QDIM_,), jnp.int32),
            pltpu.VMEM((2 * QDIM_,), jnp.float32),
            pltpu.VMEM((_RPW,), jnp.int32),
            pltpu.VMEM((4096,), jnp.int32),
            pltpu.VMEM((256,), jnp.int32),
            pltpu.VMEM((QDIM_ + _L,), jnp.int32),
            pltpu.VMEM((QDIM_ + _L,), jnp.int32),
            pltpu.SemaphoreType.DMA,
            pltpu.SemaphoreType.DMA,
        ],
    )


def kernel(x, enc_w1, enc_b1, enc_w2, enc_b2, kp_w1, kp_b1, kp_w2, kp_b2,
           kp_w3, kp_b3, k_scale):
    logits, kvec, nvec = _tc_call(
        x, enc_w1, enc_b1, enc_w2, enc_b2, kp_w1, kp_b1, kp_w2, kp_b2,
        kp_w3.reshape(1, N_HDIM_), kp_b3.reshape(1, 1), k_scale.reshape(1, 1))
    khot = _sc_mask_call()(logits, nvec)
    return khot, kvec.reshape(BATCH_, 1)


# speedup vs baseline: 17.1160x; 1.0141x over previous
import functools

import jax
import jax.numpy as jnp
import numpy as np
from jax import lax
from jax.experimental import pallas as pl
from jax.experimental.pallas import tpu as pltpu
from jax.experimental.pallas import tpu_sc as plsc

INPUT_DIM_ = 1024
N_HDIM_ = 512
QDIM_ = 4096
BATCH_ = 4096

_BM = 256
_GRID = BATCH_ // _BM
_PREC = lax.Precision.DEFAULT

_NC = 2
_NS = 16
_NW = _NC * _NS
_RPW = BATCH_ // _NW
_L = 16
_NCH = QDIM_ // _L
_SIGN = np.int32(-(2**31))


def _dot(a, b):
    return lax.dot_general(a, b, (((1,), (0,)), ((), ())),
                           precision=_PREC,
                           preferred_element_type=jnp.float32)


def _tc_body(x_ref, w1_ref, b1_ref, w2_ref, b2_ref, kpw1_ref, kpb1_ref,
             kpw2_ref, kpb2_ref, kpw3_ref, kpb3_ref, ksc_ref,
             logits_ref, k_ref, n_ref):
    x = x_ref[...]
    h = jnp.maximum(_dot(x, w1_ref[...]) + b1_ref[...][None, :], 0.0)
    logits = _dot(h, w2_ref[...]) + b2_ref[...][None, :]
    logits_ref[...] = logits
    h1 = jnp.maximum(_dot(x, kpw1_ref[0:INPUT_DIM_, :])
                     + _dot(logits, kpw1_ref[INPUT_DIM_:, :])
                     + kpb1_ref[...][None, :], 0.0)
    h2 = jnp.maximum(_dot(h1, kpw2_ref[...]) + kpb2_ref[...][None, :], 0.0)
    z = jnp.sum(h2 * kpw3_ref[...], axis=-1, keepdims=True) + kpb3_ref[...]
    k = jax.nn.sigmoid(z) * float(QDIM_)
    k = jnp.clip(k * jax.nn.sigmoid(ksc_ref[...]) * 2.0, 1.0, float(QDIM_))
    k_ref[...] = k[:, 0]
    n_ref[...] = jnp.ceil(k[:, 0]).astype(jnp.int32)


def _tc_call(x, enc_w1, enc_b1, enc_w2, enc_b2, kp_w1, kp_b1, kp_w2, kp_b2,
             kp_w3_row, kp_b3s, k_scales):
    return pl.pallas_call(
        _tc_body,
        grid=(_GRID,),
        in_specs=[
            pl.BlockSpec((_BM, INPUT_DIM_), lambda i: (i, 0)),
            pl.BlockSpec((INPUT_DIM_, N_HDIM_), lambda i: (0, 0)),
            pl.BlockSpec((N_HDIM_,), lambda i: (0,)),
            pl.BlockSpec((N_HDIM_, QDIM_), lambda i: (0, 0)),
            pl.BlockSpec((QDIM_,), lambda i: (0,)),
            pl.BlockSpec((INPUT_DIM_ + QDIM_, N_HDIM_), lambda i: (0, 0)),
            pl.BlockSpec((N_HDIM_,), lambda i: (0,)),
            pl.BlockSpec((N_HDIM_, N_HDIM_), lambda i: (0, 0)),
            pl.BlockSpec((N_HDIM_,), lambda i: (0,)),
            pl.BlockSpec((1, N_HDIM_), lambda i: (0, 0)),
            pl.BlockSpec((1, 1), lambda i: (0, 0)),
            pl.BlockSpec((1, 1), lambda i: (0, 0)),
        ],
        out_specs=[
            pl.BlockSpec((_BM, QDIM_), lambda i: (i, 0)),
            pl.BlockSpec((_BM,), lambda i: (i,)),
            pl.BlockSpec((_BM,), lambda i: (i,)),
        ],
        out_shape=[
            jax.ShapeDtypeStruct((BATCH_, QDIM_), jnp.float32),
            jax.ShapeDtypeStruct((BATCH_,), jnp.float32),
            jax.ShapeDtypeStruct((BATCH_,), jnp.int32),
        ],
    )(x, enc_w1, enc_b1, enc_w2, enc_b2, kp_w1, kp_b1, kp_w2, kp_b2,
      kp_w3_row, kp_b3s, k_scales)


def _sc_body(logits_hbm, n_hbm, out_hbm, row_v, s_v, out_v, n_v,
             hist_v, totals_v, cand_v, cand2_v, sem_in, sem_out):
    cid = lax.axis_index("c")
    sid = lax.axis_index("s")
    wid = sid * _NC + cid
    base = wid * _RPW
    pltpu.sync_copy(n_hbm.at[pl.ds(base, _RPW)], n_v)
    pltpu.async_copy(logits_hbm.at[base], row_v.at[pl.ds(0, QDIM_)], sem_in)

    lane = lax.broadcasted_iota(jnp.int32, (_L,), 0)
    laneoff = lane << 8
    zero_v = jnp.zeros((_L,), jnp.int32)
    ones_v = jnp.full((_L,), 1, jnp.int32)
    pad_v = jnp.full((_L,), _SIGN, jnp.int32)

    def zero_hist():
        @plsc.parallel_loop(0, 256, unroll=8)
        def _(ci):
            hist_v[pl.ds(ci * _L, _L)] = zero_v

    def merge_hist():
        @plsc.parallel_loop(0, 16, unroll=2)
        def _(j):
            tot = hist_v[pl.ds(j * _L, _L)]
            for reg in range(1, 16):
                tot = tot + hist_v[pl.ds(reg * 256 + j * _L, _L)]
            totals_v[pl.ds(j * _L, _L)] = tot

    def find_boundary(rem):
        @plsc.parallel_loop(0, 16, unroll=4,
                            carry=(jnp.int32(0), jnp.int32(0)))
        def fdres(j, st):
            carry, dcount = st
            tot = totals_v[pl.ds(j * _L, _L)]
            incl = plsc.cumsum(tot)
            excl = incl - tot
            cond = (excl + carry) <= rem
            dcount = dcount + plsc.all_reduce_population_count(cond)[0]
            carry = carry + incl[15]
            return carry, dcount
        D = fdres[1] - 1

        @plsc.parallel_loop(0, 16, unroll=4, carry=(zero_v, zero_v))
        def accs(j, st):
            accA, accC = st
            tot = totals_v[pl.ds(j * _L, _L)]
            dig = lane + j * _L
            accA = accA + jnp.where(dig <= D, tot, 0)
            accC = accC + jnp.where(dig == D, tot, 0)
            return accA, accC
        return D, jnp.sum(accs[0]), jnp.sum(accs[1])

    def do_row(r, carry):
        row = base + r
        ro = (r % 2) * QDIM_
        oo = (r % 2) * QDIM_
        rb = row_v.at[pl.ds(ro, QDIM_)]
        ob = out_v.at[pl.ds(oo, QDIM_)]
        pltpu.make_async_copy(logits_hbm.at[row], rb, sem_in).wait()

        @pl.when(r + 1 < _RPW)
        def _():
            pltpu.async_copy(logits_hbm.at[row + 1],
                             row_v.at[pl.ds(((r + 1) % 2) * QDIM_, QDIM_)],
                             sem_in)
        nchunk = n_v[pl.ds((r // _L) * _L, _L)]
        n = jnp.sum(jnp.where(lane == r % _L, nchunk, 0))

        zero_hist()

        @plsc.parallel_loop(0, _NCH, unroll=8)
        def _(ci):
            f = rb[pl.ds(ci * _L, _L)]
            b = plsc.bitcast(f, jnp.int32)
            sv = jnp.where(b < 0, -(b & jnp.int32(0x7FFFFFFF)), b)
            s_v[pl.ds(ci * _L, _L)] = sv
            d = lax.shift_right_logical(sv, 24) ^ 0x80
            plsc.addupdate_scatter(hist_v, [laneoff | d], ones_v)

        merge_hist()
        D, pinc, c_cand = find_boundary(QDIM_ - n)
        c_gt_b = QDIM_ - pinc
        np1 = n - c_gt_b

        @plsc.parallel_loop(0, _NCH, unroll=8, carry=jnp.int32(0))
        def off(ci, off_c):
            sv = s_v[pl.ds(ci * _L, _L)]
            d = lax.shift_right_logical(sv, 24) ^ 0x80
            m = d == D
            plsc.store_compressed(cand_v.at[pl.ds(off_c, _L)], sv, mask=m)
            return off_c + plsc.all_reduce_population_count(m)[0]
        cand_v[pl.ds(off, _L)] = pad_v
        nch2 = (c_cand + (_L - 1)) // _L
        ntot2 = nch2 * _L

        zero_hist()

        @plsc.parallel_loop(0, nch2, unroll=2)
        def _(ci):
            sv = cand_v[pl.ds(ci * _L, _L)]
            d2 = lax.shift_right_logical(sv, 16) & 0xFF
            plsc.addupdate_scatter(hist_v, [laneoff | d2], ones_v)

        merge_hist()
        D2, pinc2, _ = find_boundary(ntot2 - np1)
        c_gt_b2 = ntot2 - pinc2
        np2 = np1 - c_gt_b2

        @plsc.parallel_loop(0, nch2, unroll=2, carry=jnp.int32(0))
        def off2(ci, off_c):
            sv = cand_v[pl.ds(ci * _L, _L)]
            d2 = lax.shift_right_logical(sv, 16) & 0xFF
            m = d2 == D2
            plsc.store_compressed(cand2_v.at[pl.ds(off_c, _L)], sv, mask=m)
            return off_c + plsc.all_reduce_population_count(m)[0]
        cand2_v[pl.ds(off2, _L)] = pad_v
        nch3 = (off2 + (_L - 1)) // _L

        t0 = (D << 24) | (D2 << 16)

        def bit_step(i, t_u):
            bit = 15 - i
            t2 = t_u | (jnp.int32(1) << bit)
            thr_s = t2 ^ _SIGN

            def cc(ci, acc):
                sv = cand2_v[pl.ds(ci * _L, _L)]
                return acc + jnp.where(sv >= thr_s, 1, 0)
            cnt = jnp.sum(lax.fori_loop(0, nch3, cc, zero_v))
            return jnp.where(cnt >= np2, t2, t_u)
        t_u = lax.fori_loop(0, 16, bit_step, t0)
        thr = t_u ^ _SIGN

        def cnt_in(ci, st):
            g, e = st
            sv = cand2_v[pl.ds(ci * _L, _L)]
            g = g + jnp.where(sv > thr, 1, 0)
            e = e + jnp.where(sv == thr, 1, 0)
            return g, e
        gacc, eacc = lax.fori_loop(0, nch3, cnt_in, (zero_v, zero_v))
        c_gt = c_gt_b + c_gt_b2 + jnp.sum(gacc)
        c_eq = jnp.sum(eacc)
        rneed = n - c_gt

        @pl.when(r >= 1)
        def _():
            pltpu.make_async_copy(
                out_v.at[pl.ds(((r - 1) % 2) * QDIM_, QDIM_)],
                out_hbm.at[row - 1], sem_out).wait()

        @plsc.parallel_loop(0, _NCH, unroll=8)
        def _(ci):
            sv = s_v[pl.ds(ci * _L, _L)]
            ob[pl.ds(ci * _L, _L)] = jnp.where(sv >= thr, 1.0, 0.0)

        def fix_cond(st):
            return st[1] > 0

        def fix_body(st):
            ci, extra = st
            sv = s_v[pl.ds(ci * _L, _L)]
            m_eq = sv == thr
            eq = jnp.where(m_eq, 1, 0).astype(jnp.int32)
            ec = plsc.all_reduce_population_count(m_eq)[0]
            pref = plsc.cumsum(eq)
            clear = m_eq & (pref > (ec - extra))
            cur = ob[pl.ds(ci * _L, _L)]
            ob[pl.ds(ci * _L, _L)] = jnp.where(clear, 0.0, cur)
            return ci - 1, extra - jnp.minimum(ec, extra)
        lax.while_loop(fix_cond, fix_body,
                       (jnp.int32(_NCH - 1), c_eq - rneed))

        pltpu.async_copy(ob, out_hbm.at[row], sem_out)
        return carry
    lax.fori_loop(0, _RPW, do_row, 0)
    pltpu.make_async_copy(out_v.at[pl.ds(((_RPW - 1) % 2) * QDIM_, QDIM_)],
                          out_hbm.at[base + _RPW - 1], sem_out).wait()


@functools.cache
def _sc_mask_call():
    return pl.kernel(
        _sc_body,
        out_type=jax.ShapeDtypeStruct((BATCH_, QDIM_), jnp.float32),
        mesh=plsc.VectorSubcoreMesh(core_axis_name="c", subcore_axis_name="s",
                                    num_cores=_NC, num_subcores=_NS),
        compiler_params=pltpu.CompilerParams(needs_layout_passes=False),
        scratch_types=[
            pltpu.VMEM((2 * QDIM_,), jnp.float32),
            pltpu.VMEM((QDIM_,), jnp.int32),
            pltpu.VMEM((2 * QDIM_,), jnp.float32),
            pltpu.VMEM((_RPW,), jnp.int32),
            pltpu.VMEM((4096,), jnp.int32),
            pltpu.VMEM((256,), jnp.int32),
            pltpu.VMEM((QDIM_ + _L,), jnp.int32),
            pltpu.VMEM((QDIM_ + _L,), jnp.int32),
            pltpu.SemaphoreType.DMA,
            pltpu.SemaphoreType.DMA,
        ],
    )


def kernel(x, enc_w1, enc_b1, enc_w2, enc_b2, kp_w1, kp_b1, kp_w2, kp_b2,
           kp_w3, kp_b3, k_scale):
    logits, kvec, nvec = _tc_call(
        x, enc_w1, enc_b1, enc_w2, enc_b2, kp_w1, kp_b1, kp_w2, kp_b2,
        kp_w3.reshape(1, N_HDIM_), kp_b3.reshape(1, 1), k_scale.reshape(1, 1))
    khot = _sc_mask_call()(logits, nvec)
    return khot, kvec.reshape(BATCH_, 1)
